# baseline (speedup 1.0000x reference)
import jax
import jax.numpy as jnp
from jax import lax
from jax.experimental import pallas as pl
from jax.experimental.pallas import tpu as pltpu
from jax.experimental.pallas import tpu_sc as plsc

_NC = 2
_NS = 16


def _make_sc_kernel(R, D, rows_per_chunk, chunks_per_tile):
  n_tile = rows_per_chunk * chunks_per_tile
  mesh = plsc.VectorSubcoreMesh(core_axis_name="c", subcore_axis_name="s")

  def body(xt_hbm, idx_hbm, yt_hbm, idx0, idx1, rows0, rows1, sem0, sem1,
           wsem0, wsem1):
    cid = lax.axis_index("c")
    sid = lax.axis_index("s")
    wid = sid * _NC + cid
    base = wid * n_tile

    def idx_load(j, idx_v):
      pltpu.sync_copy(idx_hbm.at[pl.ds(base + j * rows_per_chunk,
                                       rows_per_chunk)], idx_v)

    def gather(idx_v, rows_v, sem):
      return pltpu.make_async_copy(xt_hbm.at[idx_v], rows_v, sem)

    def write(j, rows_v, wsem):
      return pltpu.make_async_copy(
          rows_v, yt_hbm.at[pl.ds(base + j * rows_per_chunk, rows_per_chunk)],
          wsem)

    nch = chunks_per_tile
    idx_load(0, idx0)
    gather(idx0, rows0, sem0).start()

    def step(i, carry):
      j = 2 * i
      idx_load(j + 1, idx1)

      @pl.when(j > 0)
      def _():
        write(j - 1, rows1, wsem1).wait()
      gather(idx1, rows1, sem1).start()
      gather(idx0, rows0, sem0).wait()
      write(j, rows0, wsem0).start()

      @pl.when(j + 2 < nch)
      def _():
        idx_load(j + 2, idx0)
      write(j, rows0, wsem0).wait()

      @pl.when(j + 2 < nch)
      def _():
        gather(idx0, rows0, sem0).start()
      gather(idx1, rows1, sem1).wait()
      write(j + 1, rows1, wsem1).start()
      return carry

    lax.fori_loop(0, nch // 2, step, 0)
    write(nch - 1, rows1, wsem1).wait()

  return pl.kernel(
      body,
      out_type=jax.ShapeDtypeStruct((R, D), jnp.float32),
      compiler_params=pltpu.CompilerParams(use_tc_tiling_on_sc=False),
      mesh=mesh,
      scratch_types=[
          pltpu.VMEM((rows_per_chunk,), jnp.int32),
          pltpu.VMEM((rows_per_chunk,), jnp.int32),
          pltpu.VMEM((rows_per_chunk, D), jnp.float32),
          pltpu.VMEM((rows_per_chunk, D), jnp.float32),
          pltpu.SemaphoreType.DMA,
          pltpu.SemaphoreType.DMA,
          pltpu.SemaphoreType.DMA,
          pltpu.SemaphoreType.DMA,
      ],
  )


def kernel(x, perms):
  B, C, H, W = x.shape
  HW = H * W
  R = C * HW
  xt = x.reshape(B, R).T
  idx = (perms.astype(jnp.int32)
         + (jnp.arange(C, dtype=jnp.int32) * HW)[:, None]).reshape(R)
  rows_per_chunk = 512
  chunks_per_tile = R // (_NC * _NS * rows_per_chunk)
  yt = _make_sc_kernel(R, B, rows_per_chunk, chunks_per_tile)(xt, idx)
  return yt.T.reshape(B, C, H, W)


# speedup vs baseline: 1.1650x; 1.1650x over previous
import jax
import jax.numpy as jnp
from jax import lax
from jax.experimental import pallas as pl
from jax.experimental.pallas import tpu as pltpu
from jax.experimental.pallas import tpu_sc as plsc

_NC = 2
_NS = 16


def _make_sc_kernel(B, C, HW):
  n_tile = HW // _NS
  b_per_core = B // _NC
  mesh = plsc.VectorSubcoreMesh(core_axis_name="c", subcore_axis_name="s")

  def body(x_hbm, perm_hbm, out_hbm, idx_v, out_v, img0, img1, sem0, sem1,
           gsem0, gsem1, wsem):
    cid = lax.axis_index("c")
    sid = lax.axis_index("s")
    base = sid * n_tile

    def stage_copy(ch, bi, img, sem):
      b = cid * b_per_core + bi
      off = b * (C * HW) + ch * HW + base
      return pltpu.make_async_copy(
          x_hbm.at[pl.ds(off, n_tile)],
          img.at[pl.ds(base, n_tile)], sem)

    half = n_tile // 2

    def gather_write(ch, bi, img):
      b = cid * b_per_core + bi
      off = b * (C * HW) + ch * HW + base
      g0 = pltpu.make_async_copy(
          img.at[idx_v.at[pl.ds(0, half)]], out_v.at[pl.ds(0, half)], gsem0)
      g1 = pltpu.make_async_copy(
          img.at[idx_v.at[pl.ds(half, half)]], out_v.at[pl.ds(half, half)],
          gsem1)
      g0.start()
      g1.start()
      g0.wait()
      w0 = pltpu.make_async_copy(
          out_v.at[pl.ds(0, half)], out_hbm.at[pl.ds(off, half)], wsem)
      w0.start()
      g1.wait()
      w1 = pltpu.make_async_copy(
          out_v.at[pl.ds(half, half)], out_hbm.at[pl.ds(off + half, half)],
          wsem)
      w1.start()
      w0.wait()
      w1.wait()

    for ch in range(C):
      pltpu.sync_copy(perm_hbm.at[pl.ds(ch * HW + base, n_tile)], idx_v)
      stage_copy(ch, 0, img0, sem0).start()
      stage_copy(ch, 1, img1, sem1).start()

      def batch_step(i, carry):
        bi = 2 * i
        stage_copy(ch, bi, img0, sem0).wait()
        plsc.subcore_barrier()
        gather_write(ch, bi, img0)
        plsc.subcore_barrier()

        @pl.when(bi + 2 < b_per_core)
        def _():
          stage_copy(ch, bi + 2, img0, sem0).start()

        stage_copy(ch, bi + 1, img1, sem1).wait()
        plsc.subcore_barrier()
        gather_write(ch, bi + 1, img1)
        plsc.subcore_barrier()

        @pl.when(bi + 3 < b_per_core)
        def _():
          stage_copy(ch, bi + 3, img1, sem1).start()

        return carry

      lax.fori_loop(0, b_per_core // 2, batch_step, 0)

  return pl.kernel(
      body,
      out_type=jax.ShapeDtypeStruct((B * C * HW,), jnp.float32),
      mesh=mesh,
      scratch_types=[
          pltpu.VMEM((n_tile,), jnp.int32),
          pltpu.VMEM((n_tile,), jnp.float32),
          pltpu.VMEM_SHARED((HW,), jnp.float32),
          pltpu.VMEM_SHARED((HW,), jnp.float32),
          pltpu.SemaphoreType.DMA,
          pltpu.SemaphoreType.DMA,
          pltpu.SemaphoreType.DMA,
          pltpu.SemaphoreType.DMA,
          pltpu.SemaphoreType.DMA,
      ],
  )


def kernel(x, perms):
  B, C, H, W = x.shape
  HW = H * W
  x_flat = x.reshape(B * C * HW)
  perms_i32 = perms.astype(jnp.int32).reshape(C * HW)
  out = _make_sc_kernel(B, C, HW)(x_flat, perms_i32)
  return out.reshape(B, C, H, W)
